# Initial kernel scaffold; baseline (speedup 1.0000x reference)
import jax
import jax.numpy as jnp
from jax.experimental import pallas as pl


def kernel(loc_data, conf_data, landm_data, priors, targets):
    raise NotImplementedError("write your pallas kernel here")


# speedup vs baseline: 33.7041x; 33.7041x over previous
import functools

import jax
import jax.numpy as jnp
from jax import lax
from jax.experimental import pallas as pl

NUM_CLASSES = 2
THRESHOLD = 0.35
NEGPOS_RATIO = 7
VAR0, VAR1 = 0.1, 0.2
N_OBJ = 8


def _smooth_l1(d):
    ad = jnp.abs(d)
    return jnp.where(ad < 1.0, 0.5 * d * d, ad - 0.5)


def _mbl_kernel(data_ref, pt_ref, tgt_ref, out_ref, *, p_real, p_pad):
    b = pl.program_id(0)

    @pl.when(b == 0)
    def _init():
        out_ref[...] = jnp.zeros_like(out_ref)

    d = data_ref[0]
    ld = d[0:4]
    x0 = d[4:5]
    x1 = d[5:6]
    lmd = d[6:16]

    pcx = pt_ref[0:1]
    pcy = pt_ref[1:2]
    pw = pt_ref[2:3]
    ph = pt_ref[3:4]
    plx = pcx - pw / 2.0
    ply = pcy - ph / 2.0
    phx = pcx + pw / 2.0
    phy = pcy + ph / 2.0

    t = tgt_ref[0]
    tx0 = t[:, 0:1]
    ty0 = t[:, 1:2]
    tx1 = t[:, 2:3]
    ty1 = t[:, 3:4]

    iw = jnp.clip(jnp.minimum(tx1, phx) - jnp.maximum(tx0, plx), 0.0, None)
    ih = jnp.clip(jnp.minimum(ty1, phy) - jnp.maximum(ty0, ply), 0.0, None)
    inter = iw * ih
    area_a = (tx1 - tx0) * (ty1 - ty0)
    area_b = (phx - plx) * (phy - ply)
    ov = inter / (area_a + area_b - inter)

    lane8 = lax.broadcasted_iota(jnp.int32, (N_OBJ, p_pad), 1)
    sub8 = lax.broadcasted_iota(jnp.int32, (N_OBJ, p_pad), 0)
    li = lax.broadcasted_iota(jnp.int32, (1, p_pad), 1)

    bpo = jnp.max(ov, axis=1, keepdims=True)
    bpi = jnp.min(jnp.where(ov == bpo, lane8, p_pad), axis=1,
                  keepdims=True)
    bto = jnp.max(ov, axis=0, keepdims=True)
    bti = jnp.min(jnp.where(ov == bto, sub8, N_OBJ), axis=0,
                  keepdims=True)

    valid = bpo >= 0.2
    m8 = bpi == li
    forced = jnp.max(jnp.where(m8 & valid, 1, 0), axis=0, keepdims=True)
    bto = jnp.where(forced > 0, 2.0, bto)
    bti_f = jnp.max(jnp.where(m8, sub8, -1), axis=0, keepdims=True)
    bti = jnp.where(bti_f >= 0, bti_f, bti)

    pos = bto >= THRESHOLD
    posf = pos.astype(jnp.float32)

    ohf = (bti == sub8).astype(jnp.float32)

    def gat(col):
        return jnp.sum(ohf * t[:, col:col + 1], axis=0, keepdims=True)

    mx0, my0, mx1, my1 = gat(0), gat(1), gat(2), gat(3)

    ecx = ((mx0 + mx1) / 2.0 - pcx) / (VAR0 * pw)
    ecy = ((my0 + my1) / 2.0 - pcy) / (VAR0 * ph)
    ew = jnp.log((mx1 - mx0) / pw) / VAR1
    eh = jnp.log((my1 - my0) / ph) / VAR1
    loc_t = jnp.concatenate([ecx, ecy, ew, eh], axis=0)
    dl = ld - loc_t
    loss_loc = jnp.sum(_smooth_l1(dl) * posf)

    lrows = []
    for k5 in range(5):
        lrows.append((gat(4 + 2 * k5) - pcx) / (VAR0 * pw))
        lrows.append((gat(5 + 2 * k5) - pcy) / (VAR0 * ph))
    landm_t = jnp.concatenate(lrows, axis=0)
    dm = lmd - landm_t
    loss_landm = jnp.sum(_smooth_l1(dm) * posf)

    mx = jnp.maximum(x0, x1)
    lse = jnp.log(jnp.exp(x0 - mx) + jnp.exp(x1 - mx)) + mx
    ce0 = lse - x0
    ce1 = lse - x1
    pos_ce = jnp.sum(ce1 * posf)

    npos_i = jnp.sum(pos.astype(jnp.int32))
    k = jnp.minimum(NEGPOS_RATIO * npos_i, p_real - 1)
    k = jnp.minimum(k, p_real - npos_i)

    padm = li >= p_real
    val = jnp.where(pos | padm, -1.0, ce0)
    key = lax.bitcast_convert_type(val, jnp.int32)
    kr = key.reshape(p_pad // 128, 128)
    vr = val.reshape(p_pad // 128, 128)

    def bs_body(_, carry):
        lo, hi = carry
        mid = lo + (hi - lo) // 2
        cnt = jnp.sum((kr > mid).astype(jnp.int32))
        ge = cnt >= k
        return jnp.where(ge, mid, lo), jnp.where(ge, hi, mid)

    lo0 = jnp.int32(-1)
    hi0 = jnp.int32(0x7F800000)
    _, t_key = lax.fori_loop(0, 31, bs_body, (lo0, hi0))
    cnt_gt = jnp.sum((kr > t_key).astype(jnp.int32))
    t_f = lax.bitcast_convert_type(t_key, jnp.float32)
    sum_gt = jnp.sum(jnp.where(kr > t_key, vr, 0.0))
    neg_ce = jnp.where(k > 0,
                       sum_gt + (k - cnt_gt).astype(jnp.float32) * t_f,
                       0.0)
    loss_cla = pos_ce + neg_ce

    npos_f = npos_i.astype(jnp.float32)
    row = jnp.concatenate([
        loss_loc.reshape(1, 1), loss_cla.reshape(1, 1),
        loss_landm.reshape(1, 1), npos_f.reshape(1, 1),
        jnp.zeros((1, 4), jnp.float32)], axis=1)
    out_ref[...] += row


def kernel(loc_data, conf_data, landm_data, priors, targets):
    batch, p_real, _ = loc_data.shape
    p_pad = pl.cdiv(p_real, 128) * 128

    data = jnp.concatenate([loc_data, conf_data, landm_data], axis=2)
    data = jnp.transpose(data, (0, 2, 1))
    data = jnp.pad(data, ((0, 0), (0, 0), (0, p_pad - p_real)))

    pad_rows = jnp.tile(jnp.array([[2.5, 2.5, 1.0, 1.0]], jnp.float32),
                        (p_pad - p_real, 1))
    priors_t = jnp.transpose(jnp.concatenate([priors, pad_rows], axis=0))

    out = pl.pallas_call(
        functools.partial(_mbl_kernel, p_real=p_real, p_pad=p_pad),
        grid=(batch,),
        in_specs=[
            pl.BlockSpec((1, 16, p_pad), lambda b: (b, 0, 0)),
            pl.BlockSpec((4, p_pad), lambda b: (0, 0)),
            pl.BlockSpec((1, N_OBJ, 15), lambda b: (b, 0, 0)),
        ],
        out_specs=pl.BlockSpec((1, 8), lambda b: (0, 0)),
        out_shape=jax.ShapeDtypeStruct((1, 8), jnp.float32),
    )(data, priors_t, targets)

    loss_loc = out[0, 0]
    loss_cla = out[0, 1]
    loss_landm = out[0, 2]
    total_pos = out[0, 3]
    n = jnp.maximum(total_pos, 1.0)
    n1 = jnp.maximum(total_pos, 1.0)
    return (loss_loc / n, loss_cla / n, loss_landm / n1)


# speedup vs baseline: 45.3926x; 1.3468x over previous
import functools

import jax
import jax.numpy as jnp
from jax import lax
from jax.experimental import pallas as pl
from jax.experimental.pallas import tpu as pltpu

NUM_CLASSES = 2
THRESHOLD = 0.35
NEGPOS_RATIO = 7
VAR0, VAR1 = 0.1, 0.2
N_OBJ = 8


def _smooth_l1(d):
    ad = jnp.abs(d)
    return jnp.where(ad < 1.0, 0.5 * d * d, ad - 0.5)


def _mbl_kernel(data_ref, pt_ref, tgt_ref, out_ref, val_ref, *,
                p_real, p_pad, batch):
    b = pl.program_id(0)

    @pl.when(b == 0)
    def _init():
        out_ref[...] = jnp.zeros_like(out_ref)

    d = data_ref[0]
    ld = d[0:4]
    x0 = d[4:5]
    x1 = d[5:6]
    lmd = d[6:16]

    pcx = pt_ref[0:1]
    pcy = pt_ref[1:2]
    pw = pt_ref[2:3]
    ph = pt_ref[3:4]
    plx = pcx - pw / 2.0
    ply = pcy - ph / 2.0
    phx = pcx + pw / 2.0
    phy = pcy + ph / 2.0

    t = tgt_ref[0]
    tx0 = t[:, 0:1]
    ty0 = t[:, 1:2]
    tx1 = t[:, 2:3]
    ty1 = t[:, 3:4]

    iw = jnp.clip(jnp.minimum(tx1, phx) - jnp.maximum(tx0, plx), 0.0, None)
    ih = jnp.clip(jnp.minimum(ty1, phy) - jnp.maximum(ty0, ply), 0.0, None)
    inter = iw * ih
    area_a = (tx1 - tx0) * (ty1 - ty0)
    area_b = (phx - plx) * (phy - ply)
    ov = inter / (area_a + area_b - inter)

    lane8 = lax.broadcasted_iota(jnp.int32, (N_OBJ, p_pad), 1)
    sub8 = lax.broadcasted_iota(jnp.int32, (N_OBJ, p_pad), 0)
    li = lax.broadcasted_iota(jnp.int32, (1, p_pad), 1)

    bpo = jnp.max(ov, axis=1, keepdims=True)
    bpi = jnp.min(jnp.where(ov == bpo, lane8, p_pad), axis=1,
                  keepdims=True)
    bto = jnp.max(ov, axis=0, keepdims=True)
    bti = jnp.min(jnp.where(ov == bto, sub8, N_OBJ), axis=0,
                  keepdims=True)

    valid = bpo >= 0.2
    m8 = bpi == li
    forced = jnp.max(jnp.where(m8 & valid, 1, 0), axis=0, keepdims=True)
    bto = jnp.where(forced > 0, 2.0, bto)
    bti_f = jnp.max(jnp.where(m8, sub8, -1), axis=0, keepdims=True)
    bti = jnp.where(bti_f >= 0, bti_f, bti)

    pos = bto >= THRESHOLD
    posf = pos.astype(jnp.float32)

    ohf = (bti == sub8).astype(jnp.float32)

    def gat(col):
        return jnp.sum(ohf * t[:, col:col + 1], axis=0, keepdims=True)

    mx0, my0, mx1, my1 = gat(0), gat(1), gat(2), gat(3)

    ecx = ((mx0 + mx1) / 2.0 - pcx) / (VAR0 * pw)
    ecy = ((my0 + my1) / 2.0 - pcy) / (VAR0 * ph)
    ew = jnp.log((mx1 - mx0) / pw) / VAR1
    eh = jnp.log((my1 - my0) / ph) / VAR1
    loc_t = jnp.concatenate([ecx, ecy, ew, eh], axis=0)
    dl = ld - loc_t
    loss_loc = jnp.sum(_smooth_l1(dl) * posf)

    lrows = []
    for k5 in range(5):
        lrows.append((gat(4 + 2 * k5) - pcx) / (VAR0 * pw))
        lrows.append((gat(5 + 2 * k5) - pcy) / (VAR0 * ph))
    landm_t = jnp.concatenate(lrows, axis=0)
    dm = lmd - landm_t
    loss_landm = jnp.sum(_smooth_l1(dm) * posf)

    mx = jnp.maximum(x0, x1)
    lse = jnp.log(jnp.exp(x0 - mx) + jnp.exp(x1 - mx)) + mx
    ce0 = lse - x0
    ce1 = lse - x1
    pos_ce = jnp.sum(ce1 * posf)

    npos_f = jnp.sum(posf)

    padm = li >= p_real
    val = jnp.where(pos | padm, -1.0, ce0)
    val_ref[pl.ds(b, 1), :] = val

    row = jnp.concatenate([
        loss_loc.reshape(1, 1), pos_ce.reshape(1, 1),
        loss_landm.reshape(1, 1), npos_f.reshape(1, 1),
        jnp.zeros((1, 4), jnp.float32)], axis=1)
    out_ref[...] += row

    @pl.when(b == batch - 1)
    def _mine():
        vals = val_ref[...]
        keys = lax.bitcast_convert_type(vals, jnp.int32)
        cnt_valid = jnp.sum((keys >= 0).astype(jnp.int32), axis=1,
                            keepdims=True)
        npos = p_real - cnt_valid
        k = jnp.minimum(NEGPOS_RATIO * npos, p_real - 1)
        k = jnp.minimum(k, cnt_valid)

        def bs_body(_, carry):
            lo, hi = carry
            mid = lo + (hi - lo) // 2
            cnt = jnp.sum((keys > mid).astype(jnp.int32), axis=1,
                          keepdims=True)
            ge = cnt >= k
            return jnp.where(ge, mid, lo), jnp.where(ge, hi, mid)

        lo0 = jnp.full((batch, 1), -1, jnp.int32)
        hi0 = jnp.full((batch, 1), 0x7F800000, jnp.int32)
        _, t_key = lax.fori_loop(0, 31, bs_body, (lo0, hi0))
        cnt_gt = jnp.sum((keys > t_key).astype(jnp.int32), axis=1,
                         keepdims=True)
        t_f = lax.bitcast_convert_type(t_key, jnp.float32)
        sum_gt = jnp.sum(jnp.where(keys > t_key, vals, 0.0), axis=1,
                         keepdims=True)
        neg = jnp.where(k > 0,
                        sum_gt + (k - cnt_gt).astype(jnp.float32) * t_f,
                        0.0)
        total = jnp.sum(neg)
        add = jnp.concatenate([
            jnp.zeros((1, 1), jnp.float32), total.reshape(1, 1),
            jnp.zeros((1, 6), jnp.float32)], axis=1)
        out_ref[...] += add


def kernel(loc_data, conf_data, landm_data, priors, targets):
    batch, p_real, _ = loc_data.shape
    p_pad = pl.cdiv(p_real, 128) * 128

    data = jnp.concatenate([loc_data, conf_data, landm_data], axis=2)
    data = jnp.transpose(data, (0, 2, 1))
    data = jnp.pad(data, ((0, 0), (0, 0), (0, p_pad - p_real)))

    pad_rows = jnp.tile(jnp.array([[2.5, 2.5, 1.0, 1.0]], jnp.float32),
                        (p_pad - p_real, 1))
    priors_t = jnp.transpose(jnp.concatenate([priors, pad_rows], axis=0))

    out = pl.pallas_call(
        functools.partial(_mbl_kernel, p_real=p_real, p_pad=p_pad,
                          batch=batch),
        grid=(batch,),
        scratch_shapes=[pltpu.VMEM((batch, p_pad), jnp.float32)],
        in_specs=[
            pl.BlockSpec((1, 16, p_pad), lambda b: (b, 0, 0)),
            pl.BlockSpec((4, p_pad), lambda b: (0, 0)),
            pl.BlockSpec((1, N_OBJ, 15), lambda b: (b, 0, 0)),
        ],
        out_specs=pl.BlockSpec((1, 8), lambda b: (0, 0)),
        out_shape=jax.ShapeDtypeStruct((1, 8), jnp.float32),
    )(data, priors_t, targets)

    loss_loc = out[0, 0]
    loss_cla = out[0, 1]
    loss_landm = out[0, 2]
    total_pos = out[0, 3]
    n = jnp.maximum(total_pos, 1.0)
    n1 = jnp.maximum(total_pos, 1.0)
    return (loss_loc / n, loss_cla / n, loss_landm / n1)


# speedup vs baseline: 55.4385x; 1.2213x over previous
import functools

import jax
import jax.numpy as jnp
from jax import lax
from jax.experimental import pallas as pl
from jax.experimental.pallas import tpu as pltpu

NUM_CLASSES = 2
THRESHOLD = 0.35
NEGPOS_RATIO = 7
VAR0, VAR1 = 0.1, 0.2
N_OBJ = 8
SUB = 8


def _smooth_l1(d):
    ad = jnp.abs(d)
    return jnp.where(ad < 1.0, 0.5 * d * d, ad - 0.5)


def _mbl_kernel(data_ref, pt_ref, tgt_ref, out_ref, val_ref, *,
                p_real, lanes, batch):
    b = pl.program_id(0)

    @pl.when(b == 0)
    def _init():
        out_ref[...] = jnp.zeros_like(out_ref)

    d = data_ref[0]
    ld = d[0:32]
    x0 = d[32:40]
    x1 = d[40:48]
    lmd = d[48:128]

    pr = pt_ref[...]
    pcx, pcy, pw, ph = pr[0:8], pr[8:16], pr[16:24], pr[24:32]
    plx = pcx - pw / 2.0
    ply = pcy - ph / 2.0
    phx = pcx + pw / 2.0
    phy = pcy + ph / 2.0
    area_b = (phx - plx) * (phy - ply)

    sub = lax.broadcasted_iota(jnp.int32, (SUB, lanes), 0)
    lane = lax.broadcasted_iota(jnp.int32, (SUB, lanes), 1)
    pidx = sub * lanes + lane

    t = tgt_ref[0]

    bto = None
    bti = None
    bpis, valids = [], []
    for j in range(N_OBJ):
        tx0, ty0, tx1, ty1 = t[j, 0], t[j, 1], t[j, 2], t[j, 3]
        iw = jnp.clip(jnp.minimum(tx1, phx) - jnp.maximum(tx0, plx), 0.0,
                      None)
        ih = jnp.clip(jnp.minimum(ty1, phy) - jnp.maximum(ty0, ply), 0.0,
                      None)
        inter = iw * ih
        area_a = (tx1 - tx0) * (ty1 - ty0)
        ov = inter / (area_a + area_b - inter)
        if j == 0:
            bto = ov
            bti = jnp.zeros((SUB, lanes), jnp.int32)
        else:
            upd = ov > bto
            bti = jnp.where(upd, j, bti)
            bto = jnp.maximum(bto, ov)
        bpo_j = jnp.max(ov)
        bpi_j = jnp.min(jnp.where(ov == bpo_j, pidx, jnp.int32(2 ** 30)))
        bpis.append(bpi_j)
        valids.append(bpo_j >= 0.2)

    for j in range(N_OBJ):
        fm = pidx == bpis[j]
        bto = jnp.where(fm & valids[j], 2.0, bto)
        bti = jnp.where(fm, j, bti)

    pos = bto >= THRESHOLD
    posf = pos.astype(jnp.float32)

    b0 = (bti & 1) != 0
    b1 = (bti & 2) != 0
    b2 = (bti & 4) != 0

    def gat(c):
        v01 = jnp.where(b0, t[1, c], t[0, c])
        v23 = jnp.where(b0, t[3, c], t[2, c])
        v45 = jnp.where(b0, t[5, c], t[4, c])
        v67 = jnp.where(b0, t[7, c], t[6, c])
        v03 = jnp.where(b1, v23, v01)
        v47 = jnp.where(b1, v67, v45)
        return jnp.where(b2, v47, v03)

    mx0, my0, mx1, my1 = gat(0), gat(1), gat(2), gat(3)

    inv_vw = 1.0 / (VAR0 * pw)
    inv_vh = 1.0 / (VAR0 * ph)

    ecx = ((mx0 + mx1) / 2.0 - pcx) * inv_vw
    ecy = ((my0 + my1) / 2.0 - pcy) * inv_vh
    ew = jnp.log((mx1 - mx0) / pw) / VAR1
    eh = jnp.log((my1 - my0) / ph) / VAR1
    loc_t = jnp.concatenate([ecx, ecy, ew, eh], axis=0)
    posf4 = jnp.concatenate([posf] * 4, axis=0)
    loss_loc = jnp.sum(_smooth_l1(ld - loc_t) * posf4)

    lrows = []
    for k5 in range(5):
        lrows.append((gat(4 + 2 * k5) - pcx) * inv_vw)
        lrows.append((gat(5 + 2 * k5) - pcy) * inv_vh)
    landm_t = jnp.concatenate(lrows, axis=0)
    posf10 = jnp.concatenate([posf] * 10, axis=0)
    loss_landm = jnp.sum(_smooth_l1(lmd - landm_t) * posf10)

    mx = jnp.maximum(x0, x1)
    lse = jnp.log(jnp.exp(x0 - mx) + jnp.exp(x1 - mx)) + mx
    ce0 = lse - x0
    ce1 = lse - x1
    pos_ce = jnp.sum(ce1 * posf)
    npos_f = jnp.sum(posf)

    padm = pidx >= p_real
    val = jnp.where(pos | padm, -1.0, ce0)
    val_ref[pl.ds(b * SUB, SUB), :] = val

    row = jnp.concatenate([
        loss_loc.reshape(1, 1), pos_ce.reshape(1, 1),
        loss_landm.reshape(1, 1), npos_f.reshape(1, 1),
        jnp.zeros((1, 4), jnp.float32)], axis=1)
    out_ref[...] += row

    @pl.when(b == batch - 1)
    def _mine():
        vals = val_ref[...]

        hp = lax.Precision.HIGHEST
        s_fold = (lax.broadcasted_iota(jnp.int32, (batch, batch * SUB), 1)
                  // SUB ==
                  lax.broadcasted_iota(jnp.int32, (batch, batch * SUB), 0)
                  ).astype(jnp.float32)
        s_spread = (lax.broadcasted_iota(jnp.int32, (batch * SUB, batch), 0)
                    // SUB ==
                    lax.broadcasted_iota(jnp.int32, (batch * SUB, batch), 1)
                    ).astype(jnp.float32)

        def fold(x):
            return lax.dot_general(s_fold, x, (((1,), (0,)), ((), ())),
                                   precision=hp)

        def spread(m):
            return lax.dot_general(s_spread, m, (((1,), (0,)), ((), ())),
                                   precision=hp)

        cnt_valid = fold(jnp.sum((vals >= 0.0).astype(jnp.float32), axis=1,
                                 keepdims=True))
        npos = p_real - cnt_valid
        k = jnp.minimum(NEGPOS_RATIO * npos, float(p_real - 1))
        k = jnp.minimum(k, cnt_valid)

        def bs_body(_, carry):
            lo, hi = carry
            mid = lo + (hi - lo) // 2
            mid_f = lax.bitcast_convert_type(jnp.maximum(mid, 0),
                                             jnp.float32)
            cnt = fold(jnp.sum((vals > spread(mid_f)).astype(jnp.float32),
                               axis=1, keepdims=True))
            ge = cnt >= k
            return jnp.where(ge, mid, lo), jnp.where(ge, hi, mid)

        lo0 = jnp.full((batch, 1), -1, jnp.int32)
        hi0 = jnp.full((batch, 1), 0x7F800000, jnp.int32)
        _, t_key = lax.fori_loop(0, 31, bs_body, (lo0, hi0))
        t_key = jnp.clip(t_key, 0, 0x7F7FFFFF)
        t_f = lax.bitcast_convert_type(t_key, jnp.float32)
        gtmask = vals > spread(t_f)
        cnt_gt = fold(jnp.sum(gtmask.astype(jnp.float32), axis=1,
                              keepdims=True))
        sum_gt = fold(jnp.sum(jnp.where(gtmask, vals, 0.0), axis=1,
                              keepdims=True))
        neg = jnp.where(k > 0.0, sum_gt + (k - cnt_gt) * t_f,
                        0.0)
        total = jnp.sum(neg)
        add = jnp.concatenate([
            jnp.zeros((1, 1), jnp.float32), total.reshape(1, 1),
            jnp.zeros((1, 6), jnp.float32)], axis=1)
        out_ref[...] += add


def kernel(loc_data, conf_data, landm_data, priors, targets):
    batch, p_real, _ = loc_data.shape
    lanes = pl.cdiv(p_real, SUB * 128) * 128
    p_pad = SUB * lanes

    data = jnp.concatenate([loc_data, conf_data, landm_data], axis=2)
    data = jnp.transpose(data, (0, 2, 1))
    data = jnp.pad(data, ((0, 0), (0, 0), (0, p_pad - p_real)))
    data = data.reshape(batch, 16 * SUB, lanes)

    pad_rows = jnp.tile(jnp.array([[2.5, 2.5, 1.0, 1.0]], jnp.float32),
                        (p_pad - p_real, 1))
    priors_t = jnp.transpose(jnp.concatenate([priors, pad_rows], axis=0))
    priors_t = priors_t.reshape(4 * SUB, lanes)

    out = pl.pallas_call(
        functools.partial(_mbl_kernel, p_real=p_real, lanes=lanes,
                          batch=batch),
        grid=(batch,),
        scratch_shapes=[pltpu.VMEM((batch * SUB, lanes), jnp.float32)],
        in_specs=[
            pl.BlockSpec((1, 16 * SUB, lanes), lambda b: (b, 0, 0)),
            pl.BlockSpec((4 * SUB, lanes), lambda b: (0, 0)),
            pl.BlockSpec((1, N_OBJ, 15), lambda b: (b, 0, 0)),
        ],
        out_specs=pl.BlockSpec((1, 8), lambda b: (0, 0)),
        out_shape=jax.ShapeDtypeStruct((1, 8), jnp.float32),
    )(data, priors_t, targets)

    loss_loc = out[0, 0]
    loss_cla = out[0, 1]
    loss_landm = out[0, 2]
    n = jnp.maximum(out[0, 3], 1.0)
    return (loss_loc / n, loss_cla / n, loss_landm / n)


# speedup vs baseline: 72.7805x; 1.3128x over previous
import functools

import jax
import jax.numpy as jnp
from jax import lax
from jax.experimental import pallas as pl
from jax.experimental.pallas import tpu as pltpu

NUM_CLASSES = 2
THRESHOLD = 0.35
NEGPOS_RATIO = 7
VAR0, VAR1 = 0.1, 0.2
N_OBJ = 8
SUB = 8


def _smooth_l1(d):
    ad = jnp.abs(d)
    return jnp.where(ad < 1.0, 0.5 * d * d, ad - 0.5)


def _mbl_kernel(loc_ref, conf_ref, lmd_ref, pt_ref, tgt_ref, out_ref,
                val_ref, acc_ref, *, p_real, lanes, batch, rps):
    b = pl.program_id(0)

    @pl.when(b == 0)
    def _init2():
        acc_ref[...] = jnp.zeros_like(acc_ref)

    part = _one_row(loc_ref, conf_ref, lmd_ref, pt_ref, tgt_ref, val_ref,
                    b, 0, p_real=p_real, lanes=lanes, rps=rps)
    acc_ref[...] += part

    @pl.when(b == batch // rps - 1)
    def _fin():
        a1 = jnp.sum(acc_ref[...], axis=1, keepdims=True)
        loc_s = jnp.sum(a1[0:8])
        landm_s = jnp.sum(a1[8:16])
        ce_s = jnp.sum(a1[16:24])
        npos_s = jnp.sum(a1[24:32])
        total = _mine_all(val_ref, p_real=p_real, batch=batch)
        out_ref[...] = jnp.concatenate([
            loc_s.reshape(1, 1), (ce_s + total).reshape(1, 1),
            landm_s.reshape(1, 1), npos_s.reshape(1, 1),
            jnp.zeros((1, 4), jnp.float32)], axis=1)


def _one_row(loc_ref, conf_ref, lmd_ref, pt_ref, tgt_ref, val_ref, b, r,
             *, p_real, lanes, rps):
    ld = loc_ref[r]
    cf = conf_ref[r]
    x0 = cf[0:8]
    x1 = cf[8:16]
    lmd = lmd_ref[r]

    pr = pt_ref[...]
    pcx, pcy, pw, ph = pr[0:8], pr[8:16], pr[16:24], pr[24:32]
    plx = pcx - pw / 2.0
    ply = pcy - ph / 2.0
    phx = pcx + pw / 2.0
    phy = pcy + ph / 2.0
    area_b = (phx - plx) * (phy - ply)

    sub = lax.broadcasted_iota(jnp.int32, (SUB, lanes), 0)
    lane = lax.broadcasted_iota(jnp.int32, (SUB, lanes), 1)
    pidx = sub * lanes + lane

    t = tgt_ref[r]

    bto = None
    bti = None
    ovl = []
    for j in range(N_OBJ):
        tx0, ty0, tx1, ty1 = t[j, 0], t[j, 1], t[j, 2], t[j, 3]
        iw = jnp.clip(jnp.minimum(tx1, phx) - jnp.maximum(tx0, plx), 0.0,
                      None)
        ih = jnp.clip(jnp.minimum(ty1, phy) - jnp.maximum(ty0, ply), 0.0,
                      None)
        inter = iw * ih
        area_a = (tx1 - tx0) * (ty1 - ty0)
        ov = inter / (area_a + area_b - inter)
        ovl.append(ov)
        if j == 0:
            bto = ov
            bti = jnp.zeros((SUB, lanes), jnp.int32)
        else:
            upd = ov > bto
            bti = jnp.where(upd, j, bti)
            bto = jnp.maximum(bto, ov)

    ovs = jnp.concatenate(ovl, axis=0)
    m1 = jnp.max(ovs, axis=1, keepdims=True)
    bpos = [jnp.max(m1[SUB * j:SUB * j + SUB]) for j in range(N_OBJ)]
    bporows = jnp.concatenate(
        [jnp.broadcast_to(p.reshape(1, 1), (SUB, 1)) for p in bpos], axis=0)
    cand = jnp.where(ovs == bporows,
                     jnp.concatenate([pidx] * N_OBJ, axis=0),
                     jnp.int32(2 ** 30))
    m2 = jnp.min(cand, axis=1, keepdims=True)
    bpis = [jnp.min(m2[SUB * j:SUB * j + SUB]) for j in range(N_OBJ)]
    valids = [p >= 0.2 for p in bpos]

    for j in range(N_OBJ):
        fm = pidx == bpis[j]
        bto = jnp.where(fm & valids[j], 2.0, bto)
        bti = jnp.where(fm, j, bti)

    pos = bto >= THRESHOLD
    posf = pos.astype(jnp.float32)

    b0 = (bti & 1) != 0
    b1 = (bti & 2) != 0
    b2 = (bti & 4) != 0

    def gat(c):
        v01 = jnp.where(b0, t[1, c], t[0, c])
        v23 = jnp.where(b0, t[3, c], t[2, c])
        v45 = jnp.where(b0, t[5, c], t[4, c])
        v67 = jnp.where(b0, t[7, c], t[6, c])
        v03 = jnp.where(b1, v23, v01)
        v47 = jnp.where(b1, v67, v45)
        return jnp.where(b2, v47, v03)

    mx0, my0, mx1, my1 = gat(0), gat(1), gat(2), gat(3)

    inv_vw = 1.0 / (VAR0 * pw)
    inv_vh = 1.0 / (VAR0 * ph)

    ecx = ((mx0 + mx1) / 2.0 - pcx) * inv_vw
    ecy = ((my0 + my1) / 2.0 - pcy) * inv_vh
    ew = jnp.log((mx1 - mx0) / pw) / VAR1
    eh = jnp.log((my1 - my0) / ph) / VAR1
    loc_t = jnp.concatenate([ecx, ecy, ew, eh], axis=0)
    sl = _smooth_l1(ld - loc_t)
    locp = (sl[0:8] + sl[8:16] + sl[16:24] + sl[24:32]) * posf

    lrows = []
    for k5 in range(5):
        lrows.append((gat(4 + 2 * k5) - pcx) * inv_vw)
        lrows.append((gat(5 + 2 * k5) - pcy) * inv_vh)
    landm_t = jnp.concatenate(lrows, axis=0)
    sm = _smooth_l1(lmd - landm_t)
    smf = sm[0:8]
    for g in range(1, 10):
        smf = smf + sm[8 * g:8 * g + 8]
    lmp = smf * posf

    mx = jnp.maximum(x0, x1)
    lse = jnp.log(jnp.exp(x0 - mx) + jnp.exp(x1 - mx)) + mx
    ce0 = lse - x0
    cep = (lse - x1) * posf

    padm = pidx >= p_real
    val = jnp.where(pos | padm, -1.0, ce0)
    val_ref[pl.ds((b * rps + r) * SUB, SUB), :] = val

    return jnp.concatenate([locp, lmp, cep, posf], axis=0)


def _mine_all(val_ref, *, p_real, batch):
    vals = val_ref[...]

    hp = lax.Precision.HIGHEST
    s_fold = (lax.broadcasted_iota(jnp.int32, (batch, batch * SUB), 1)
              // SUB ==
              lax.broadcasted_iota(jnp.int32, (batch, batch * SUB), 0)
              ).astype(jnp.float32)
    s_spread = (lax.broadcasted_iota(jnp.int32, (batch * SUB, batch), 0)
                // SUB ==
                lax.broadcasted_iota(jnp.int32, (batch * SUB, batch), 1)
                ).astype(jnp.float32)

    def fold(x):
        return lax.dot_general(s_fold, x, (((1,), (0,)), ((), ())),
                               precision=hp)

    def spread(m):
        return lax.dot_general(s_spread, m, (((1,), (0,)), ((), ())),
                               precision=hp)

    cnt_valid = fold(jnp.sum((vals >= 0.0).astype(jnp.float32), axis=1,
                             keepdims=True))
    npos = p_real - cnt_valid
    k = jnp.minimum(NEGPOS_RATIO * npos, float(p_real - 1))
    k = jnp.minimum(k, cnt_valid)

    def bs_body(_, carry):
        lo, hi = carry
        mid = lo + (hi - lo) // 2
        mid_f = lax.bitcast_convert_type(jnp.maximum(mid, 0),
                                         jnp.float32)
        cnt = fold(jnp.sum((vals > spread(mid_f)).astype(jnp.float32),
                           axis=1, keepdims=True))
        ge = cnt >= k
        return jnp.where(ge, mid, lo), jnp.where(ge, hi, mid)

    lo0 = jnp.full((batch, 1), -1, jnp.int32)
    hi0 = jnp.full((batch, 1), 0x7F800000, jnp.int32)
    _, t_key = lax.fori_loop(0, 31, bs_body, (lo0, hi0))
    t_key = jnp.clip(t_key, 0, 0x7F7FFFFF)
    t_f = lax.bitcast_convert_type(t_key, jnp.float32)
    gtmask = vals > spread(t_f)
    cnt_gt = fold(jnp.sum(gtmask.astype(jnp.float32), axis=1,
                          keepdims=True))
    sum_gt = fold(jnp.sum(jnp.where(gtmask, vals, 0.0), axis=1,
                          keepdims=True))
    neg = jnp.where(k > 0.0, sum_gt + (k - cnt_gt) * t_f,
                    0.0)
    return jnp.sum(neg)


def kernel(loc_data, conf_data, landm_data, priors, targets):
    batch, p_real, _ = loc_data.shape
    lanes = pl.cdiv(p_real, SUB * 128) * 128
    p_pad = SUB * lanes

    def prep(x, c):
        x = jnp.transpose(x, (0, 2, 1))
        x = jnp.pad(x, ((0, 0), (0, 0), (0, p_pad - p_real)))
        return x.reshape(batch, c * SUB, lanes)

    loc_p = prep(loc_data, 4)
    conf_p = prep(conf_data, NUM_CLASSES)
    lmd_p = prep(landm_data, 10)

    pad_rows = jnp.tile(jnp.array([[2.5, 2.5, 1.0, 1.0]], jnp.float32),
                        (p_pad - p_real, 1))
    priors_t = jnp.transpose(jnp.concatenate([priors, pad_rows], axis=0))
    priors_t = priors_t.reshape(4 * SUB, lanes)

    rps = 1
    out = pl.pallas_call(
        functools.partial(_mbl_kernel, p_real=p_real, lanes=lanes,
                          batch=batch, rps=rps),
        grid=(batch // rps,),
        scratch_shapes=[pltpu.VMEM((batch * SUB, lanes), jnp.float32),
                        pltpu.VMEM((4 * SUB, lanes), jnp.float32)],
        in_specs=[
            pl.BlockSpec((rps, 4 * SUB, lanes), lambda b: (b, 0, 0)),
            pl.BlockSpec((rps, NUM_CLASSES * SUB, lanes),
                         lambda b: (b, 0, 0)),
            pl.BlockSpec((rps, 10 * SUB, lanes), lambda b: (b, 0, 0)),
            pl.BlockSpec((4 * SUB, lanes), lambda b: (0, 0)),
            pl.BlockSpec((rps, N_OBJ, 15), lambda b: (b, 0, 0)),
        ],
        out_specs=pl.BlockSpec((1, 8), lambda b: (0, 0)),
        out_shape=jax.ShapeDtypeStruct((1, 8), jnp.float32),
    )(loc_p, conf_p, lmd_p, priors_t, targets)

    loss_loc = out[0, 0]
    loss_cla = out[0, 1]
    loss_landm = out[0, 2]
    n = jnp.maximum(out[0, 3], 1.0)
    return (loss_loc / n, loss_cla / n, loss_landm / n)
